# baseline (speedup 1.0000x reference)
import jax
import jax.numpy as jnp
from jax.experimental import pallas as pl


TN = 1024


def _split3_bf16(x):
    p1 = x.astype(jnp.bfloat16).astype(jnp.float32)
    r1 = x - p1
    p2 = r1.astype(jnp.bfloat16).astype(jnp.float32)
    p3 = r1 - p2
    return p1, p2, p3


def _chamfer_kernel(a1_ref, b1_ref, a2_ref, b2_ref, asq_ref, bsq_ref,
                    d1_ref, d2_ref):
    b = pl.program_id(0)
    i = pl.program_id(1)
    ni = pl.num_programs(1)

    f = jax.lax.dot_general(
        a1_ref[0], b1_ref[0], (((1,), (1,)), ((), ())),
        preferred_element_type=jnp.float32)
    part2 = jnp.min(f, axis=0)[None, :]

    et = jax.lax.dot_general(
        a2_ref[0], b2_ref[0], (((1,), (1,)), ((), ())),
        preferred_element_type=jnp.float32)
    part1 = jnp.min(et, axis=0)
    d1_ref[pl.ds(b, 1), pl.ds(i * TN, TN)] = jnp.maximum(
        part1 + asq_ref[b, pl.ds(i * TN, TN)], 0.0)[None, :]

    @pl.when(i == 0)
    def _():
        d2_ref[pl.ds(b, 1), :] = part2

    @pl.when(i != 0)
    def _():
        d2_ref[pl.ds(b, 1), :] = jnp.minimum(d2_ref[pl.ds(b, 1), :], part2)

    @pl.when(i == ni - 1)
    def _():
        d2_ref[pl.ds(b, 1), :] = jnp.maximum(
            d2_ref[pl.ds(b, 1), :] + bsq_ref[pl.ds(b, 1), :], 0.0)


@jax.jit
def kernel(xyz1, xyz2):
    B, N, _ = xyz1.shape
    M = xyz2.shape[1]
    a_sq = jnp.sum(xyz1 * xyz1, axis=2)
    b_sq = jnp.sum(xyz2 * xyz2, axis=2)
    nb = -2.0 * xyz2
    g1, g2, g3 = _split3_bf16(a_sq)
    h1, h2, h3 = _split3_bf16(b_sq)
    ones_n = jnp.ones((B, N, 1), jnp.float32)
    ones_m = jnp.ones((B, M, 1), jnp.float32)
    st = lambda *xs: jnp.concatenate(
        [x[..., None] for x in xs], axis=2)
    a1 = jnp.concatenate([xyz1, st(g1, g2, g3)], axis=2)
    b1 = jnp.concatenate([nb, ones_m, ones_m, ones_m], axis=2)
    a2 = jnp.concatenate([nb, st(h1, h2, h3)], axis=2)
    b2 = jnp.concatenate([xyz1, ones_n, ones_n, ones_n], axis=2)
    grid = (B, N // TN)
    d1, d2 = pl.pallas_call(
        _chamfer_kernel,
        grid=grid,
        in_specs=[
            pl.BlockSpec((1, TN, 6), lambda b, i: (b, i, 0)),
            pl.BlockSpec((1, M, 6), lambda b, i: (b, 0, 0)),
            pl.BlockSpec((1, M, 6), lambda b, i: (b, 0, 0)),
            pl.BlockSpec((1, TN, 6), lambda b, i: (b, i, 0)),
            pl.BlockSpec((B, N), lambda b, i: (0, 0)),
            pl.BlockSpec((B, M), lambda b, i: (0, 0)),
        ],
        out_specs=[
            pl.BlockSpec((B, N), lambda b, i: (0, 0)),
            pl.BlockSpec((B, M), lambda b, i: (0, 0)),
        ],
        out_shape=[
            jax.ShapeDtypeStruct((B, N), jnp.float32),
            jax.ShapeDtypeStruct((B, M), jnp.float32),
        ],
    )(a1, b1, a2, b2, a_sq, b_sq)
    return (d1, d2)


# speedup vs baseline: 1.5218x; 1.5218x over previous
import jax
import jax.numpy as jnp
from jax.experimental import pallas as pl


TN = 1024


def _chamfer_kernel(x1_ref, x2_ref, d1_ref, d2_ref):
    b = pl.program_id(0)
    i = pl.program_id(1)
    a = x1_ref[0]
    c = x2_ref[0]
    a_sq = jnp.sum(a * a, axis=1)
    c_sq = jnp.sum(c * c, axis=1) * 0.25
    nc = jax.lax.dot_general(
        a, c, (((1,), (1,)), ((), ())),
        preferred_element_type=jnp.float32)
    d = (a_sq[:, None] + c_sq[None, :]) + nc
    d1_ref[pl.ds(b, 1), pl.ds(i * TN, TN)] = jnp.maximum(
        jnp.min(d, axis=1), 0.0)[None, :]
    part2 = jnp.min(d, axis=0)[None, :]

    @pl.when(i == 0)
    def _():
        d2_ref[pl.ds(b, 1), :] = part2

    @pl.when(i != 0)
    def _():
        d2_ref[pl.ds(b, 1), :] = jnp.minimum(d2_ref[pl.ds(b, 1), :], part2)


@jax.jit
def kernel(xyz1, xyz2):
    B, N, _ = xyz1.shape
    M = xyz2.shape[1]
    grid = (B, N // TN)
    d1, d2 = pl.pallas_call(
        _chamfer_kernel,
        grid=grid,
        in_specs=[
            pl.BlockSpec((1, TN, 3), lambda b, i: (b, i, 0)),
            pl.BlockSpec((1, M, 3), lambda b, i: (b, 0, 0)),
        ],
        out_specs=[
            pl.BlockSpec((B, N), lambda b, i: (0, 0)),
            pl.BlockSpec((B, M), lambda b, i: (0, 0)),
        ],
        out_shape=[
            jax.ShapeDtypeStruct((B, N), jnp.float32),
            jax.ShapeDtypeStruct((B, M), jnp.float32),
        ],
    )(xyz1, -2.0 * xyz2)
    d2 = jnp.maximum(d2, 0.0)
    return (d1, d2)


# speedup vs baseline: 1.6320x; 1.0725x over previous
import jax
import jax.numpy as jnp
from jax.experimental import pallas as pl


TN = 1024


def _chamfer_kernel(x1_ref, x2_ref, asq_ref, csq_ref, d1_ref, d2_ref):
    b = pl.program_id(0)
    i = pl.program_id(1)
    a = x1_ref[0]
    c = x2_ref[0]
    a_sq = asq_ref[b, pl.ds(i * TN, TN)]
    c_sq = csq_ref[b, :]
    nc = jax.lax.dot_general(
        a, c, (((1,), (1,)), ((), ())),
        preferred_element_type=jnp.float32)
    e = nc + c_sq[None, :]
    d1_ref[pl.ds(b, 1), pl.ds(i * TN, TN)] = jnp.maximum(
        jnp.min(e, axis=1) + a_sq, 0.0)[None, :]
    part2 = jnp.min(e + a_sq[:, None], axis=0)[None, :]

    @pl.when(i == 0)
    def _():
        d2_ref[pl.ds(b, 1), :] = part2

    @pl.when(i != 0)
    def _():
        d2_ref[pl.ds(b, 1), :] = jnp.minimum(d2_ref[pl.ds(b, 1), :], part2)


@jax.jit
def kernel(xyz1, xyz2):
    B, N, _ = xyz1.shape
    M = xyz2.shape[1]
    a_sq = jnp.sum(xyz1 * xyz1, axis=2)
    b_sq = jnp.sum(xyz2 * xyz2, axis=2)
    grid = (B, N // TN)
    d1, d2 = pl.pallas_call(
        _chamfer_kernel,
        grid=grid,
        in_specs=[
            pl.BlockSpec((1, TN, 3), lambda b, i: (b, i, 0)),
            pl.BlockSpec((1, M, 3), lambda b, i: (b, 0, 0)),
            pl.BlockSpec((B, N), lambda b, i: (0, 0)),
            pl.BlockSpec((B, M), lambda b, i: (0, 0)),
        ],
        out_specs=[
            pl.BlockSpec((B, N), lambda b, i: (0, 0)),
            pl.BlockSpec((B, M), lambda b, i: (0, 0)),
        ],
        out_shape=[
            jax.ShapeDtypeStruct((B, N), jnp.float32),
            jax.ShapeDtypeStruct((B, M), jnp.float32),
        ],
    )(xyz1, -2.0 * xyz2, a_sq, b_sq)
    d2 = jnp.maximum(d2, 0.0)
    return (d1, d2)
